# baseline (speedup 1.0000x reference)
import functools

import jax
import jax.numpy as jnp
from jax import lax
from jax.experimental import pallas as pl
from jax.experimental.pallas import tpu as pltpu
from jax.experimental.pallas import tpu_sc as plsc

TOPK = 4
NEG = -1e30


def _simt_body(x_ref, keys_ref, hasher_ref, simt_ref):
    ht = lax.dot_general(hasher_ref[...], x_ref[...],
                         (((1,), (1,)), ((), ())),
                         preferred_element_type=jnp.float32)
    simt_ref[...] = jnp.dot(keys_ref[...], ht,
                            preferred_element_type=jnp.float32)


def _compute_simt(x2, keys, hasher_w):
    t, d = x2.shape
    p, pd = keys.shape
    nt = t // 256
    return pl.pallas_call(
        _simt_body,
        grid=(nt,),
        in_specs=[
            pl.BlockSpec((256, d), lambda i: (i, 0)),
            pl.BlockSpec((p, pd), lambda i: (0, 0)),
            pl.BlockSpec((pd, d), lambda i: (0, 0)),
        ],
        out_specs=pl.BlockSpec((p, 256), lambda i: (0, i)),
        out_shape=jax.ShapeDtypeStruct((p, t), jnp.float32),
    )(x2, keys, hasher_w)


def _topk_stats_sc(simt):
    p, t = simt.shape
    lanes = 16
    slab_w = 128
    n_slabs = t // slab_w
    n_cg = slab_w // lanes

    mesh = plsc.VectorSubcoreMesh(core_axis_name="c", subcore_axis_name="s")

    @functools.partial(
        pl.kernel,
        mesh=mesh,
        out_type=[
            jax.ShapeDtypeStruct((t,), jnp.float32),
            jax.ShapeDtypeStruct((t,), jnp.float32),
            jax.ShapeDtypeStruct((t,), jnp.float32),
        ],
        scratch_types=[
            pltpu.VMEM((p, slab_w), jnp.float32),
            pltpu.VMEM((slab_w,), jnp.float32),
            pltpu.VMEM((slab_w,), jnp.float32),
            pltpu.VMEM((slab_w,), jnp.float32),
        ],
    )
    def topk_kernel(simt_hbm, m1_hbm, m4_hbm, z_hbm, slab_v, o1_v, o4_v, oz_v):
        wid = lax.axis_index("s") * 2 + lax.axis_index("c")

        @pl.when(wid < n_slabs)
        def _work():
            base = wid * slab_w
            pltpu.sync_copy(simt_hbm.at[:, pl.ds(base, slab_w)], slab_v)

            neg = jnp.full((lanes,), NEG, jnp.float32)

            def body(r, carry):
                out = []
                for cg in range(n_cg):
                    m1, m2, m3, m4 = carry[cg]
                    row = slab_v[r, pl.ds(cg * lanes, lanes)]
                    a1 = jnp.maximum(m1, row)
                    r2 = jnp.minimum(m1, row)
                    a2 = jnp.maximum(m2, r2)
                    r3 = jnp.minimum(m2, r2)
                    a3 = jnp.maximum(m3, r3)
                    r4 = jnp.minimum(m3, r3)
                    a4 = jnp.maximum(m4, r4)
                    out.append((a1, a2, a3, a4))
                return tuple(out)

            init = tuple((neg, neg, neg, neg) for _ in range(n_cg))
            stats = lax.fori_loop(0, p, body, init)
            for cg in range(n_cg):
                m1, m2, m3, m4 = stats[cg]
                z = (1.0 + jnp.exp(m2 - m1) + jnp.exp(m3 - m1)
                     + jnp.exp(m4 - m1))
                sl = pl.ds(cg * lanes, lanes)
                o1_v[sl] = m1
                o4_v[sl] = m4
                oz_v[sl] = z
            pltpu.sync_copy(o1_v, m1_hbm.at[pl.ds(base, slab_w)])
            pltpu.sync_copy(o4_v, m4_hbm.at[pl.ds(base, slab_w)])
            pltpu.sync_copy(oz_v, z_hbm.at[pl.ds(base, slab_w)])

    return topk_kernel(simt)


def _moe_body(x_ref, keys_ref, vdt_ref, vur_ref, hasher_ref, m1_ref, m4_ref,
              z_ref, scale_ref, out_ref, g3_s, *, n_eblk, pd):
    e = pl.program_id(1)
    e_blk = g3_s.shape[2]

    @pl.when(e == 0)
    def _gates():
        h = lax.dot_general(x_ref[...], hasher_ref[...],
                            (((1,), (1,)), ((), ())),
                            preferred_element_type=jnp.float32)
        sim = lax.dot_general(h, keys_ref[...],
                              (((1,), (1,)), ((), ())),
                              preferred_element_type=jnp.float32)
        g = jnp.where(sim >= m4_ref[...],
                      jnp.exp(sim - m1_ref[...]) / z_ref[...],
                      0.0)
        for eb in range(n_eblk):
            g3_s[eb] = g[:, eb * e_blk:(eb + 1) * e_blk]

    row = lax.broadcasted_iota(jnp.int32, (e_blk, e_blk * pd), 0)
    col = lax.broadcasted_iota(jnp.int32, (e_blk, e_blk * pd), 1)
    expand = (row == col // pd).astype(jnp.float32)
    g_exp = jnp.dot(g3_s[e], expand, preferred_element_type=jnp.float32)

    hidden = jnp.dot(x_ref[...], vdt_ref[...],
                     preferred_element_type=jnp.float32)
    act = hidden * (1.0 / (1.0 + jnp.exp(-hidden)))
    contrib = jnp.dot(g_exp * act, vur_ref[...],
                      preferred_element_type=jnp.float32)
    scale = scale_ref[0, 0]

    @pl.when(e == 0)
    def _init():
        out_ref[...] = x_ref[...] + scale * contrib

    @pl.when(e != 0)
    def _acc():
        out_ref[...] += scale * contrib


def kernel(x, keys, values_down, values_up, hasher_w, scale):
    b, t, d = x.shape
    p, pd = keys.shape
    x2 = x.reshape(t, d)
    vdt = values_down.transpose(1, 0, 2).reshape(d, p * pd)
    vur = values_up.reshape(p * pd, d)
    scale_arr = jnp.reshape(scale, (1, 1))

    simt = _compute_simt(x2, keys, hasher_w)
    m1, m4, z = _topk_stats_sc(simt)
    m1c = m1.reshape(t, 1)
    m4c = m4.reshape(t, 1)
    zc = z.reshape(t, 1)

    n_eblk = 4
    e_blk = p // n_eblk
    t_blk = min(512, t)
    n_tblk = t // t_blk

    grid = (n_tblk, n_eblk)
    out = pl.pallas_call(
        functools.partial(_moe_body, n_eblk=n_eblk, pd=pd),
        grid=grid,
        in_specs=[
            pl.BlockSpec((t_blk, d), lambda ti, ei: (ti, 0)),
            pl.BlockSpec((p, pd), lambda ti, ei: (0, 0)),
            pl.BlockSpec((d, e_blk * pd), lambda ti, ei: (0, ei)),
            pl.BlockSpec((e_blk * pd, d), lambda ti, ei: (ei, 0)),
            pl.BlockSpec((pd, d), lambda ti, ei: (0, 0)),
            pl.BlockSpec((t_blk, 1), lambda ti, ei: (ti, 0)),
            pl.BlockSpec((t_blk, 1), lambda ti, ei: (ti, 0)),
            pl.BlockSpec((t_blk, 1), lambda ti, ei: (ti, 0)),
            pl.BlockSpec(memory_space=pltpu.SMEM),
        ],
        out_specs=pl.BlockSpec((t_blk, d), lambda ti, ei: (ti, 0)),
        out_shape=jax.ShapeDtypeStruct((t, d), jnp.float32),
        scratch_shapes=[
            pltpu.VMEM((n_eblk, t_blk, e_blk), jnp.float32),
        ],
    )(x2, keys, vdt, vur, hasher_w, m1c, m4c, zc, scale_arr)
    return out.reshape(b, t, d)


# speedup vs baseline: 1.0218x; 1.0218x over previous
import functools

import jax
import jax.numpy as jnp
from jax import lax
from jax.experimental import pallas as pl
from jax.experimental.pallas import tpu as pltpu
from jax.experimental.pallas import tpu_sc as plsc

TOPK = 4
NEG = -1e30


def _simt_body(x_ref, keys_ref, hasher_ref, simt_ref):
    ht = lax.dot_general(hasher_ref[...], x_ref[...],
                         (((1,), (1,)), ((), ())),
                         preferred_element_type=jnp.float32)
    simt_ref[...] = jnp.dot(keys_ref[...], ht,
                            preferred_element_type=jnp.float32)


def _compute_simt(x2, keys, hasher_w):
    t, d = x2.shape
    p, pd = keys.shape
    return pl.pallas_call(
        _simt_body,
        grid=(1,),
        in_specs=[
            pl.BlockSpec((t, d), lambda i: (0, 0)),
            pl.BlockSpec((p, pd), lambda i: (0, 0)),
            pl.BlockSpec((pd, d), lambda i: (0, 0)),
        ],
        out_specs=pl.BlockSpec((p, t), lambda i: (0, 0)),
        out_shape=jax.ShapeDtypeStruct((p, t), jnp.float32),
    )(x2, keys, hasher_w)


def _topk_stats_sc(simt):
    p, t = simt.shape
    lanes = 16
    slab_w = 128
    n_slabs = t // slab_w
    n_cg = slab_w // lanes

    mesh = plsc.VectorSubcoreMesh(core_axis_name="c", subcore_axis_name="s")

    @functools.partial(
        pl.kernel,
        mesh=mesh,
        out_type=[
            jax.ShapeDtypeStruct((t,), jnp.float32),
            jax.ShapeDtypeStruct((t,), jnp.float32),
            jax.ShapeDtypeStruct((t,), jnp.float32),
        ],
        scratch_types=[
            pltpu.VMEM((p, slab_w), jnp.float32),
            pltpu.VMEM((slab_w,), jnp.float32),
            pltpu.VMEM((slab_w,), jnp.float32),
            pltpu.VMEM((slab_w,), jnp.float32),
        ],
    )
    def topk_kernel(simt_hbm, m1_hbm, m4_hbm, z_hbm, slab_v, o1_v, o4_v, oz_v):
        wid = lax.axis_index("s") * 2 + lax.axis_index("c")

        @pl.when(wid < n_slabs)
        def _work():
            base = wid * slab_w
            pltpu.sync_copy(simt_hbm.at[:, pl.ds(base, slab_w)], slab_v)

            neg = jnp.full((lanes,), NEG, jnp.float32)

            def body(r, carry):
                out = []
                for cg in range(n_cg):
                    m1, m2, m3, m4 = carry[cg]
                    row = slab_v[r, pl.ds(cg * lanes, lanes)]
                    a1 = jnp.maximum(m1, row)
                    r2 = jnp.minimum(m1, row)
                    a2 = jnp.maximum(m2, r2)
                    r3 = jnp.minimum(m2, r2)
                    a3 = jnp.maximum(m3, r3)
                    r4 = jnp.minimum(m3, r3)
                    a4 = jnp.maximum(m4, r4)
                    out.append((a1, a2, a3, a4))
                return tuple(out)

            init = tuple((neg, neg, neg, neg) for _ in range(n_cg))
            stats = lax.fori_loop(0, p, body, init)
            for cg in range(n_cg):
                m1, m2, m3, m4 = stats[cg]
                z = (1.0 + jnp.exp(m2 - m1) + jnp.exp(m3 - m1)
                     + jnp.exp(m4 - m1))
                sl = pl.ds(cg * lanes, lanes)
                o1_v[sl] = m1
                o4_v[sl] = m4
                oz_v[sl] = z
            pltpu.sync_copy(o1_v, m1_hbm.at[pl.ds(base, slab_w)])
            pltpu.sync_copy(o4_v, m4_hbm.at[pl.ds(base, slab_w)])
            pltpu.sync_copy(oz_v, z_hbm.at[pl.ds(base, slab_w)])

    return topk_kernel(simt)


def _moe_body(x_ref, keys_ref, vdt_ref, vur_ref, hasher_ref, m1_ref, m4_ref,
              z_ref, scale_ref, out_ref, g3_s, *, n_eblk, pd):
    e = pl.program_id(1)
    e_blk = g3_s.shape[2]

    @pl.when(e == 0)
    def _gates():
        h = lax.dot_general(x_ref[...], hasher_ref[...],
                            (((1,), (1,)), ((), ())),
                            preferred_element_type=jnp.float32)
        sim = lax.dot_general(h, keys_ref[...],
                              (((1,), (1,)), ((), ())),
                              preferred_element_type=jnp.float32)
        g = jnp.where(sim >= m4_ref[...],
                      jnp.exp(sim - m1_ref[...]) / z_ref[...],
                      0.0)
        for eb in range(n_eblk):
            g3_s[eb] = g[:, eb * e_blk:(eb + 1) * e_blk]

    row = lax.broadcasted_iota(jnp.int32, (e_blk, e_blk * pd), 0)
    col = lax.broadcasted_iota(jnp.int32, (e_blk, e_blk * pd), 1)
    expand = (row == col // pd).astype(jnp.float32)
    g_exp = jnp.dot(g3_s[e], expand, preferred_element_type=jnp.float32)

    hidden = jnp.dot(x_ref[...], vdt_ref[...],
                     preferred_element_type=jnp.float32)
    act = hidden * (1.0 / (1.0 + jnp.exp(-hidden)))
    contrib = jnp.dot(g_exp * act, vur_ref[...],
                      preferred_element_type=jnp.float32)
    scale = scale_ref[0, 0]

    @pl.when(e == 0)
    def _init():
        out_ref[...] = x_ref[...] + scale * contrib

    @pl.when(e != 0)
    def _acc():
        out_ref[...] += scale * contrib


def kernel(x, keys, values_down, values_up, hasher_w, scale):
    b, t, d = x.shape
    p, pd = keys.shape
    x2 = x.reshape(t, d)
    vdt = values_down.transpose(1, 0, 2).reshape(d, p * pd)
    vur = values_up.reshape(p * pd, d)
    scale_arr = jnp.reshape(scale, (1, 1))

    simt = _compute_simt(x2, keys, hasher_w)
    m1, m4, z = _topk_stats_sc(simt)
    m1c = m1.reshape(t, 1)
    m4c = m4.reshape(t, 1)
    zc = z.reshape(t, 1)

    n_eblk = 4
    e_blk = p // n_eblk
    t_blk = min(512, t)
    n_tblk = t // t_blk

    grid = (n_tblk, n_eblk)
    out = pl.pallas_call(
        functools.partial(_moe_body, n_eblk=n_eblk, pd=pd),
        grid=grid,
        in_specs=[
            pl.BlockSpec((t_blk, d), lambda ti, ei: (ti, 0)),
            pl.BlockSpec((p, pd), lambda ti, ei: (0, 0)),
            pl.BlockSpec((d, e_blk * pd), lambda ti, ei: (0, ei)),
            pl.BlockSpec((e_blk * pd, d), lambda ti, ei: (ei, 0)),
            pl.BlockSpec((pd, d), lambda ti, ei: (0, 0)),
            pl.BlockSpec((t_blk, 1), lambda ti, ei: (ti, 0)),
            pl.BlockSpec((t_blk, 1), lambda ti, ei: (ti, 0)),
            pl.BlockSpec((t_blk, 1), lambda ti, ei: (ti, 0)),
            pl.BlockSpec(memory_space=pltpu.SMEM),
        ],
        out_specs=pl.BlockSpec((t_blk, d), lambda ti, ei: (ti, 0)),
        out_shape=jax.ShapeDtypeStruct((t, d), jnp.float32),
        scratch_shapes=[
            pltpu.VMEM((n_eblk, t_blk, e_blk), jnp.float32),
        ],
    )(x2, keys, vdt, vur, hasher_w, m1c, m4c, zc, scale_arr)
    return out.reshape(b, t, d)


# speedup vs baseline: 1.0383x; 1.0161x over previous
import functools

import jax
import jax.numpy as jnp
from jax import lax
from jax.experimental import pallas as pl
from jax.experimental.pallas import tpu as pltpu
from jax.experimental.pallas import tpu_sc as plsc

TOPK = 4
NEG = -1e30


def _simt_body(x_ref, keys_ref, hasher_ref, simt_ref):
    ht = lax.dot_general(hasher_ref[...], x_ref[...],
                         (((1,), (1,)), ((), ())),
                         preferred_element_type=jnp.float32)
    simt_ref[...] = jnp.dot(keys_ref[...], ht,
                            preferred_element_type=jnp.float32)


def _compute_simt(x2, keys, hasher_w):
    t, d = x2.shape
    p, pd = keys.shape
    return pl.pallas_call(
        _simt_body,
        grid=(1,),
        in_specs=[
            pl.BlockSpec((t, d), lambda i: (0, 0)),
            pl.BlockSpec((p, pd), lambda i: (0, 0)),
            pl.BlockSpec((pd, d), lambda i: (0, 0)),
        ],
        out_specs=pl.BlockSpec((p, t), lambda i: (0, 0)),
        out_shape=jax.ShapeDtypeStruct((p, t), jnp.float32),
    )(x2, keys, hasher_w)


def _topk_stats_sc(simt):
    p, t = simt.shape
    lanes = 16
    slab_w = 128
    n_slabs = t // slab_w
    n_cg = slab_w // lanes

    mesh = plsc.VectorSubcoreMesh(core_axis_name="c", subcore_axis_name="s")

    @functools.partial(
        pl.kernel,
        mesh=mesh,
        out_type=[
            jax.ShapeDtypeStruct((t,), jnp.float32),
            jax.ShapeDtypeStruct((t,), jnp.float32),
            jax.ShapeDtypeStruct((t,), jnp.float32),
        ],
        scratch_types=[
            pltpu.VMEM((p, slab_w), jnp.float32),
            pltpu.VMEM((slab_w,), jnp.float32),
            pltpu.VMEM((slab_w,), jnp.float32),
            pltpu.VMEM((slab_w,), jnp.float32),
        ],
    )
    def topk_kernel(simt_hbm, m1_hbm, m4_hbm, z_hbm, slab_v, o1_v, o4_v, oz_v):
        wid = lax.axis_index("s") * 2 + lax.axis_index("c")

        @pl.when(wid < n_slabs)
        def _work():
            base = wid * slab_w
            pltpu.sync_copy(simt_hbm.at[:, pl.ds(base, slab_w)], slab_v)

            neg = jnp.full((lanes,), NEG, jnp.float32)

            def insert(carry_cg, row):
                m1, m2, m3, m4 = carry_cg
                a1 = jnp.maximum(m1, row)
                r2 = jnp.minimum(m1, row)
                a2 = jnp.maximum(m2, r2)
                r3 = jnp.minimum(m2, r2)
                a3 = jnp.maximum(m3, r3)
                r4 = jnp.minimum(m3, r3)
                a4 = jnp.maximum(m4, r4)
                return (a1, a2, a3, a4)

            def body(i, carry):
                out = []
                r = i * 2
                for cg in range(n_cg):
                    c = insert(carry[cg], slab_v[r, pl.ds(cg * lanes, lanes)])
                    c = insert(c, slab_v[r + 1, pl.ds(cg * lanes, lanes)])
                    out.append(c)
                return tuple(out)

            init = tuple((neg, neg, neg, neg) for _ in range(n_cg))
            stats = lax.fori_loop(0, p // 2, body, init)
            for cg in range(n_cg):
                m1, m2, m3, m4 = stats[cg]
                z = (1.0 + jnp.exp(m2 - m1) + jnp.exp(m3 - m1)
                     + jnp.exp(m4 - m1))
                sl = pl.ds(cg * lanes, lanes)
                o1_v[sl] = m1
                o4_v[sl] = m4
                oz_v[sl] = z
            pltpu.sync_copy(o1_v, m1_hbm.at[pl.ds(base, slab_w)])
            pltpu.sync_copy(o4_v, m4_hbm.at[pl.ds(base, slab_w)])
            pltpu.sync_copy(oz_v, z_hbm.at[pl.ds(base, slab_w)])

    return topk_kernel(simt)


def _moe_body(x_ref, keys_ref, vdt_ref, vur_ref, hasher_ref, m1_ref, m4_ref,
              z_ref, scale_ref, out_ref, g3_s, *, n_eblk, pd):
    e = pl.program_id(1)
    e_blk = g3_s.shape[2]

    @pl.when(e == 0)
    def _gates():
        h = lax.dot_general(x_ref[...], hasher_ref[...],
                            (((1,), (1,)), ((), ())),
                            preferred_element_type=jnp.float32)
        sim = lax.dot_general(h, keys_ref[...],
                              (((1,), (1,)), ((), ())),
                              preferred_element_type=jnp.float32)
        g = jnp.where(sim >= m4_ref[...],
                      jnp.exp(sim - m1_ref[...]) / z_ref[...],
                      0.0)
        for eb in range(n_eblk):
            g3_s[eb] = g[:, eb * e_blk:(eb + 1) * e_blk]

    row = lax.broadcasted_iota(jnp.int32, (e_blk, e_blk * pd), 0)
    col = lax.broadcasted_iota(jnp.int32, (e_blk, e_blk * pd), 1)
    expand = (row == col // pd).astype(jnp.float32)
    g_exp = jnp.dot(g3_s[e], expand, preferred_element_type=jnp.float32)

    hidden = jnp.dot(x_ref[...], vdt_ref[...],
                     preferred_element_type=jnp.float32)
    act = hidden * (1.0 / (1.0 + jnp.exp(-hidden)))
    contrib = jnp.dot(g_exp * act, vur_ref[...],
                      preferred_element_type=jnp.float32)
    scale = scale_ref[0, 0]

    @pl.when(e == 0)
    def _init():
        out_ref[...] = x_ref[...] + scale * contrib

    @pl.when(e != 0)
    def _acc():
        out_ref[...] += scale * contrib


def kernel(x, keys, values_down, values_up, hasher_w, scale):
    b, t, d = x.shape
    p, pd = keys.shape
    x2 = x.reshape(t, d)
    vdt = values_down.transpose(1, 0, 2).reshape(d, p * pd)
    vur = values_up.reshape(p * pd, d)
    scale_arr = jnp.reshape(scale, (1, 1))

    simt = _compute_simt(x2, keys, hasher_w)
    m1, m4, z = _topk_stats_sc(simt)
    m1c = m1.reshape(t, 1)
    m4c = m4.reshape(t, 1)
    zc = z.reshape(t, 1)

    n_eblk = 8
    e_blk = p // n_eblk
    t_blk = min(1024, t)
    n_tblk = t // t_blk

    grid = (n_tblk, n_eblk)
    out = pl.pallas_call(
        functools.partial(_moe_body, n_eblk=n_eblk, pd=pd),
        grid=grid,
        in_specs=[
            pl.BlockSpec((t_blk, d), lambda ti, ei: (ti, 0)),
            pl.BlockSpec((p, pd), lambda ti, ei: (0, 0)),
            pl.BlockSpec((d, e_blk * pd), lambda ti, ei: (0, ei)),
            pl.BlockSpec((e_blk * pd, d), lambda ti, ei: (ei, 0)),
            pl.BlockSpec((pd, d), lambda ti, ei: (0, 0)),
            pl.BlockSpec((t_blk, 1), lambda ti, ei: (ti, 0)),
            pl.BlockSpec((t_blk, 1), lambda ti, ei: (ti, 0)),
            pl.BlockSpec((t_blk, 1), lambda ti, ei: (ti, 0)),
            pl.BlockSpec(memory_space=pltpu.SMEM),
        ],
        out_specs=pl.BlockSpec((t_blk, d), lambda ti, ei: (ti, 0)),
        out_shape=jax.ShapeDtypeStruct((t, d), jnp.float32),
        scratch_shapes=[
            pltpu.VMEM((n_eblk, t_blk, e_blk), jnp.float32),
        ],
    )(x2, keys, vdt, vur, hasher_w, m1c, m4c, zc, scale_arr)
    return out.reshape(b, t, d)
